# baseline (speedup 1.0000x reference)
import functools

import jax
import jax.numpy as jnp
from jax import lax
from jax.experimental import pallas as pl
from jax.experimental.pallas import tpu as pltpu
from jax.experimental.pallas import tpu_sc as plsc


_BS = 2048


def _tc_body(x_ref, pos_ref, out_ref):
    out_ref[0, :, :] = x_ref[0, :, :] + pos_ref[:, :]


def _kernel_tc(x, pos_table):
    batch, seq_len, d_model = x.shape
    nb = seq_len // _BS
    return pl.pallas_call(
        _tc_body,
        grid=(nb, batch),
        in_specs=[
            pl.BlockSpec((1, _BS, d_model), lambda i, j: (j, i, 0)),
            pl.BlockSpec((_BS, d_model), lambda i, j: (i, 0)),
        ],
        out_specs=pl.BlockSpec((1, _BS, d_model), lambda i, j: (j, i, 0)),
        out_shape=jax.ShapeDtypeStruct(x.shape, x.dtype),
    )(x, pos_table[:seq_len])


_NC = 2
_NS = 16
_NW = _NC * _NS

_D = 1024
_SEQ = 2048
_BATCH = 4
_ROWS_PER_W = _SEQ // _NW
_CHUNK_ROWS = 16
_CHUNK = _CHUNK_ROWS * _D
_NCHUNK_PER_B = _ROWS_PER_W // _CHUNK_ROWS
_STRIP = _ROWS_PER_W * _D
_LANES = 16


_NSBUF = 2


def _make_sc_body(n_batch):
    def _sc_body(x_hbm, pos_hbm, out_hbm, pos_v, xb0, xb1, sp, si0, si1, so0, so1):
        wid = lax.axis_index("s") * _NC + lax.axis_index("c")
        row0 = wid * _ROWS_PER_W

        bufs = (xb0, xb1)
        in_sems = (si0, si1)
        out_sems = (so0, so1)

        chunks = []
        for b in range(n_batch):
            for c in range(_NCHUNK_PER_B):
                chunks.append((b, c))
        n = len(chunks)

        def x_slice(i):
            b, c = chunks[i]
            return pl.ds(b * _SEQ + row0 + c * _CHUNK_ROWS, _CHUNK_ROWS)

        out_copies = [None, None]
        in_copy = [None, None]

        in_copy[0] = pltpu.async_copy(x_hbm.at[x_slice(0)], bufs[0], in_sems[0])
        in_copy[1] = pltpu.async_copy(x_hbm.at[x_slice(1)], bufs[1], in_sems[1])
        pos_copy = pltpu.async_copy(pos_hbm.at[pl.ds(row0, _ROWS_PER_W)], pos_v, sp)

        for i in range(n):
            k = i % 2
            in_copy[k].wait()
            if i == 0:
                pos_copy.wait()
            if out_copies[k] is not None:
                out_copies[k].wait()
                out_copies[k] = None

            xb = bufs[k]
            pos_row0 = chunks[i][1] * _CHUNK_ROWS

            @plsc.parallel_loop(0, _CHUNK, _LANES, unroll=8)
            def _add(off, xb=xb, pos_row0=pos_row0):
                r = lax.shift_right_logical(off, 10)
                cc = pl.multiple_of(lax.bitwise_and(off, _D - 1), _LANES)
                plsc.addupdate(
                    xb.at[r, pl.ds(cc, _LANES)], pos_v[pos_row0 + r, pl.ds(cc, _LANES)]
                )

            out_copies[k] = pltpu.async_copy(xb, out_hbm.at[x_slice(i)], out_sems[k])
            if i + 2 < n:
                in_copy[k] = pltpu.async_copy(x_hbm.at[x_slice(i + 2)], bufs[k], in_sems[k])

        for oc in out_copies:
            if oc is not None:
                oc.wait()

    return _sc_body


def _kernel_sc(x, pos_table):
    batch, seq_len, d_model = x.shape
    x2 = x.reshape(batch * seq_len, d_model)
    mesh = plsc.VectorSubcoreMesh(core_axis_name="c", subcore_axis_name="s")
    out2 = pl.kernel(
        _make_sc_body(batch),
        out_type=jax.ShapeDtypeStruct((batch * seq_len, d_model), jnp.float32),
        mesh=mesh,
        scratch_types=[
            pltpu.VMEM((_ROWS_PER_W, _D), jnp.float32),
            pltpu.VMEM((_CHUNK_ROWS, _D), jnp.float32),
            pltpu.VMEM((_CHUNK_ROWS, _D), jnp.float32),
            pltpu.SemaphoreType.DMA,
            pltpu.SemaphoreType.DMA,
            pltpu.SemaphoreType.DMA,
            pltpu.SemaphoreType.DMA,
            pltpu.SemaphoreType.DMA,
        ],
    )(x2, pos_table)
    return out2.reshape(x.shape)


def kernel(x, pos_table):
    tc_part = _kernel_tc(x[:3], pos_table)
    sc_part = _kernel_sc(x[3:], pos_table)
    return jnp.concatenate([tc_part, sc_part], axis=0)


# speedup vs baseline: 1.7457x; 1.7457x over previous
import jax
import jax.numpy as jnp
from jax import lax
from jax.experimental import pallas as pl
from jax.experimental.pallas import tpu as pltpu
from jax.experimental.pallas import tpu_sc as plsc

_NC = 2
_NS = 16
_NW = _NC * _NS

_D = 1024
_SEQ = 2048
_ROWS_PER_W = _SEQ // _NW
_CHUNK_ROWS = 16
_CHUNK = _CHUNK_ROWS * _D
_NCHUNK_PER_B = _ROWS_PER_W // _CHUNK_ROWS
_LANES = 16


def _make_sc_body(n_batch):
    def _sc_body(x_hbm, pos_hbm, out_hbm, pos_v, xb0, xb1, sp, si0, si1, so0, so1):
        wid = lax.axis_index("s") * _NC + lax.axis_index("c")
        row0 = wid * _ROWS_PER_W

        bufs = (xb0, xb1)
        in_sems = (si0, si1)
        out_sems = (so0, so1)

        chunks = []
        for b in range(n_batch):
            for c in range(_NCHUNK_PER_B):
                chunks.append((b, c))
        n = len(chunks)

        def x_slice(i):
            b, c = chunks[i]
            return pl.ds(b * _SEQ + row0 + c * _CHUNK_ROWS, _CHUNK_ROWS)

        out_copies = [None, None]
        in_copy = [None, None]

        in_copy[0] = pltpu.async_copy(x_hbm.at[x_slice(0)], bufs[0], in_sems[0])
        in_copy[1] = pltpu.async_copy(x_hbm.at[x_slice(1)], bufs[1], in_sems[1])
        pos_copy = pltpu.async_copy(pos_hbm.at[pl.ds(row0, _ROWS_PER_W)], pos_v, sp)

        for i in range(n):
            k = i % 2
            in_copy[k].wait()
            if i == 0:
                pos_copy.wait()
            if out_copies[k] is not None:
                out_copies[k].wait()
                out_copies[k] = None

            xb = bufs[k]
            pos_row0 = chunks[i][1] * _CHUNK_ROWS

            @plsc.parallel_loop(0, _CHUNK, _LANES, unroll=8)
            def _add(off, xb=xb, pos_row0=pos_row0):
                r = lax.shift_right_logical(off, 10)
                cc = pl.multiple_of(lax.bitwise_and(off, _D - 1), _LANES)
                plsc.addupdate(
                    xb.at[r, pl.ds(cc, _LANES)], pos_v[pos_row0 + r, pl.ds(cc, _LANES)]
                )

            out_copies[k] = pltpu.async_copy(xb, out_hbm.at[x_slice(i)], out_sems[k])
            if i + 2 < n:
                in_copy[k] = pltpu.async_copy(x_hbm.at[x_slice(i + 2)], bufs[k], in_sems[k])

        for oc in out_copies:
            if oc is not None:
                oc.wait()

    return _sc_body


def kernel(x, pos_table):
    batch, seq_len, d_model = x.shape
    x2 = x.reshape(batch * seq_len, d_model)
    mesh = plsc.VectorSubcoreMesh(core_axis_name="c", subcore_axis_name="s")
    out2 = pl.kernel(
        _make_sc_body(batch),
        out_type=jax.ShapeDtypeStruct((batch * seq_len, d_model), jnp.float32),
        mesh=mesh,
        scratch_types=[
            pltpu.VMEM((_ROWS_PER_W, _D), jnp.float32),
            pltpu.VMEM((_CHUNK_ROWS, _D), jnp.float32),
            pltpu.VMEM((_CHUNK_ROWS, _D), jnp.float32),
            pltpu.SemaphoreType.DMA,
            pltpu.SemaphoreType.DMA,
            pltpu.SemaphoreType.DMA,
            pltpu.SemaphoreType.DMA,
            pltpu.SemaphoreType.DMA,
        ],
    )(x2, pos_table)
    return out2.reshape(x.shape)


# speedup vs baseline: 1.8342x; 1.0507x over previous
import jax
import jax.numpy as jnp
from jax import lax
from jax.experimental import pallas as pl
from jax.experimental.pallas import tpu as pltpu
from jax.experimental.pallas import tpu_sc as plsc

_NC = 2
_NS = 16
_NW = _NC * _NS

_D = 1024
_SEQ = 2048
_ROWS_PER_W = _SEQ // _NW
_CHUNK_ROWS = 16
_CHUNK = _CHUNK_ROWS * _D
_NCHUNK_PER_B = _ROWS_PER_W // _CHUNK_ROWS
_LANES = 16


def _make_sc_body(n_batch):
    def _sc_body(x_hbm, pos_hbm, out_hbm, pos_v, xb0, xb1, xb2, sp,
                 si0, si1, si2, so0, so1, so2):
        wid = lax.axis_index("s") * _NC + lax.axis_index("c")
        row0 = wid * _ROWS_PER_W

        bufs = (xb0, xb1, xb2)
        in_sems = (si0, si1, si2)
        out_sems = (so0, so1, so2)

        chunks = []
        for b in range(n_batch):
            for c in range(_NCHUNK_PER_B):
                chunks.append((b, c))
        n = len(chunks)

        def x_slice(i):
            b, c = chunks[i]
            return pl.ds(b * _SEQ + row0 + c * _CHUNK_ROWS, _CHUNK_ROWS)

        out_copies = [None, None, None]
        in_copy = [None, None, None]

        in_copy[0] = pltpu.async_copy(x_hbm.at[x_slice(0)], bufs[0], in_sems[0])
        in_copy[1] = pltpu.async_copy(x_hbm.at[x_slice(1)], bufs[1], in_sems[1])
        in_copy[2] = pltpu.async_copy(x_hbm.at[x_slice(2)], bufs[2], in_sems[2])
        pos_copy = pltpu.async_copy(pos_hbm.at[pl.ds(row0, _ROWS_PER_W)], pos_v, sp)

        for i in range(n):
            k = i % 3
            in_copy[k].wait()
            if i == 0:
                pos_copy.wait()
            if out_copies[k] is not None:
                out_copies[k].wait()
                out_copies[k] = None

            xb = bufs[k]
            pos_row0 = chunks[i][1] * _CHUNK_ROWS

            @plsc.parallel_loop(0, _CHUNK, _LANES, unroll=8)
            def _add(off, xb=xb, pos_row0=pos_row0):
                r = lax.shift_right_logical(off, 10)
                cc = pl.multiple_of(lax.bitwise_and(off, _D - 1), _LANES)
                plsc.addupdate(
                    xb.at[r, pl.ds(cc, _LANES)], pos_v[pos_row0 + r, pl.ds(cc, _LANES)]
                )

            out_copies[k] = pltpu.async_copy(xb, out_hbm.at[x_slice(i)], out_sems[k])
            if i + 3 < n:
                in_copy[k] = pltpu.async_copy(x_hbm.at[x_slice(i + 3)], bufs[k], in_sems[k])

        for oc in out_copies:
            if oc is not None:
                oc.wait()

    return _sc_body


def kernel(x, pos_table):
    batch, seq_len, d_model = x.shape
    x2 = x.reshape(batch * seq_len, d_model)
    mesh = plsc.VectorSubcoreMesh(core_axis_name="c", subcore_axis_name="s")
    out2 = pl.kernel(
        _make_sc_body(batch),
        out_type=jax.ShapeDtypeStruct((batch * seq_len, d_model), jnp.float32),
        mesh=mesh,
        scratch_types=[
            pltpu.VMEM((_ROWS_PER_W, _D), jnp.float32),
            pltpu.VMEM((_CHUNK_ROWS, _D), jnp.float32),
            pltpu.VMEM((_CHUNK_ROWS, _D), jnp.float32),
            pltpu.VMEM((_CHUNK_ROWS, _D), jnp.float32),
            pltpu.SemaphoreType.DMA,
            pltpu.SemaphoreType.DMA,
            pltpu.SemaphoreType.DMA,
            pltpu.SemaphoreType.DMA,
            pltpu.SemaphoreType.DMA,
            pltpu.SemaphoreType.DMA,
            pltpu.SemaphoreType.DMA,
        ],
    )(x2, pos_table)
    return out2.reshape(x.shape)
